# Initial kernel scaffold; baseline (speedup 1.0000x reference)
import jax
import jax.numpy as jnp
from jax.experimental import pallas as pl


def kernel(waveforms):
    raise NotImplementedError("write your pallas kernel here")


# speedup vs baseline: 1.1519x; 1.1519x over previous
import functools

import jax
import jax.numpy as jnp
from jax import lax
from jax.experimental import pallas as pl
from jax.experimental.pallas import tpu as pltpu
from jax.experimental.pallas import tpu_sc as plsc

DROP_LEN_LOW = 1000
DROP_LEN_HIGH = 2000
DROP_CNT_LOW = 1
DROP_CNT_HIGH = 5
P = 0.5
MAX_DROPS = DROP_CNT_HIGH
LANES = 16
SPAN_PAD = 16


def _drop_spans(batch, length):
    key = jax.random.key(42)
    k_apply, k_cnt, k_len, k_start = jax.random.split(key, 4)
    apply_flag = jax.random.uniform(k_apply, ()) <= P
    n_drops = jax.random.randint(k_cnt, (batch,), DROP_CNT_LOW, DROP_CNT_HIGH + 1)
    drop_len = jax.random.randint(
        k_len, (batch, MAX_DROPS), DROP_LEN_LOW, min(DROP_LEN_HIGH + 1, length)
    )
    u = jax.random.uniform(k_start, (batch, MAX_DROPS))
    start_range = jnp.maximum(1, length - drop_len)
    start = jnp.floor(u * start_range).astype(jnp.int32)
    active = (jnp.arange(MAX_DROPS)[None, :] < n_drops[:, None]) & apply_flag
    starts = jnp.where(active, start, 0).astype(jnp.int32)
    ends = jnp.where(active, start + drop_len, 0).astype(jnp.int32)
    return starts, ends


@functools.lru_cache(maxsize=None)
def _build(batch, length):
    info = plsc.get_sparse_core_info()
    nc, ns = info.num_cores, info.num_subcores
    nw = nc * ns
    assert batch == nw, (batch, nw)

    chunk = 16000
    assert length % chunk == 0
    nch = length // chunk

    mesh = plsc.VectorSubcoreMesh(core_axis_name="c", subcore_axis_name="s")

    @functools.partial(
        pl.kernel,
        mesh=mesh,
        out_type=jax.ShapeDtypeStruct((batch, length), jnp.float32),
        scratch_types=[
            pltpu.VMEM((SPAN_PAD,), jnp.int32),
            pltpu.VMEM((SPAN_PAD,), jnp.int32),
            pltpu.VMEM((chunk,), jnp.float32),
        ],
    )
    def sc_fn(w_hbm, sb_hbm, eb_hbm, out_hbm, s_vm, e_vm, buf):
        r = lax.axis_index("s") * nc + lax.axis_index("c")
        pltpu.sync_copy(sb_hbm.at[r], s_vm)
        pltpu.sync_copy(eb_hbm.at[r], e_vm)
        sv = s_vm[...]
        ev = e_vm[...]
        iota16 = lax.iota(jnp.int32, LANES)
        for ci in range(nch):
            c = ci * chunk
            pltpu.sync_copy(w_hbm.at[r, pl.ds(c, chunk)], buf)
            for i in range(MAX_DROPS):
                s_i = sv[i]
                e_i = ev[i]
                lo = jnp.maximum(s_i, c)
                hi = jnp.minimum(e_i, c + chunk)
                jlo = jnp.maximum((lo - c) // LANES, 0)
                jhi = jnp.minimum((hi - c + LANES - 1) // LANES, chunk // LANES)
                jhi = jnp.maximum(jlo, jhi)

                def vb(j, carry, s_i=s_i, e_i=e_i, c=c):
                    off = j * LANES
                    t = (c + off) + iota16
                    x = buf[pl.ds(off, LANES)]
                    d = (t >= s_i) & (t < e_i)
                    buf[pl.ds(off, LANES)] = jnp.where(d, jnp.float32(0.0), x)
                    return carry

                lax.fori_loop(jlo, jhi, vb, 0)
            pltpu.sync_copy(buf, out_hbm.at[r, pl.ds(c, chunk)])

    return sc_fn


def kernel(waveforms):
    b, ch, length = waveforms.shape
    batch = b * ch
    starts, ends = _drop_spans(batch, length)
    sb = jnp.zeros((batch, SPAN_PAD), jnp.int32).at[:, :MAX_DROPS].set(starts)
    eb = jnp.zeros((batch, SPAN_PAD), jnp.int32).at[:, :MAX_DROPS].set(ends)
    fn = _build(batch, length)
    out = fn(waveforms.reshape(batch, length), sb, eb)
    return out.reshape(b, ch, length)


# speedup vs baseline: 1.2333x; 1.0707x over previous
import functools

import jax
import jax.numpy as jnp
from jax import lax
from jax.experimental import pallas as pl
from jax.experimental.pallas import tpu as pltpu
from jax.experimental.pallas import tpu_sc as plsc

DROP_LEN_LOW = 1000
DROP_LEN_HIGH = 2000
DROP_CNT_LOW = 1
DROP_CNT_HIGH = 5
P = 0.5
MAX_DROPS = DROP_CNT_HIGH
LANES = 16
SPAN_PAD = 16


def _drop_spans(batch, length):
    key = jax.random.key(42)
    k_apply, k_cnt, k_len, k_start = jax.random.split(key, 4)
    apply_flag = jax.random.uniform(k_apply, ()) <= P
    n_drops = jax.random.randint(k_cnt, (batch,), DROP_CNT_LOW, DROP_CNT_HIGH + 1)
    drop_len = jax.random.randint(
        k_len, (batch, MAX_DROPS), DROP_LEN_LOW, min(DROP_LEN_HIGH + 1, length)
    )
    u = jax.random.uniform(k_start, (batch, MAX_DROPS))
    start_range = jnp.maximum(1, length - drop_len)
    start = jnp.floor(u * start_range).astype(jnp.int32)
    active = (jnp.arange(MAX_DROPS)[None, :] < n_drops[:, None]) & apply_flag
    starts = jnp.where(active, start, 0).astype(jnp.int32)
    ends = jnp.where(active, start + drop_len, 0).astype(jnp.int32)
    return starts, ends


@functools.lru_cache(maxsize=None)
def _build(batch, length):
    info = plsc.get_sparse_core_info()
    nc, ns = info.num_cores, info.num_subcores
    nw = nc * ns
    assert batch == nw, (batch, nw)

    chunk = 16000
    nbuf = 4
    assert length % chunk == 0
    nch = length // chunk

    mesh = plsc.VectorSubcoreMesh(core_axis_name="c", subcore_axis_name="s")

    @functools.partial(
        pl.kernel,
        mesh=mesh,
        out_type=jax.ShapeDtypeStruct((batch, length), jnp.float32),
        scratch_types=[
            pltpu.VMEM((SPAN_PAD,), jnp.int32),
            pltpu.VMEM((SPAN_PAD,), jnp.int32),
        ]
        + [pltpu.VMEM((chunk,), jnp.float32) for _ in range(nbuf)]
        + [pltpu.SemaphoreType.DMA for _ in range(2 * nbuf)],
    )
    def sc_fn(w_hbm, sb_hbm, eb_hbm, out_hbm, s_vm, e_vm, *rest):
        bufs = rest[:nbuf]
        in_sems = rest[nbuf : 2 * nbuf]
        out_sems = rest[2 * nbuf : 3 * nbuf]
        r = lax.axis_index("s") * nc + lax.axis_index("c")
        pltpu.sync_copy(sb_hbm.at[r], s_vm)
        pltpu.sync_copy(eb_hbm.at[r], e_vm)
        sv = s_vm[...]
        ev = e_vm[...]
        iota16 = lax.iota(jnp.int32, LANES)

        def start_in(ci):
            b = ci % nbuf
            return pltpu.async_copy(
                w_hbm.at[r, pl.ds(ci * chunk, chunk)], bufs[b], in_sems[b]
            )

        def start_out(ci):
            b = ci % nbuf
            return pltpu.async_copy(
                bufs[b], out_hbm.at[r, pl.ds(ci * chunk, chunk)], out_sems[b]
            )

        def mask_chunk(ci):
            b = ci % nbuf
            buf = bufs[b]
            c = ci * chunk
            for i in range(MAX_DROPS):
                s_i = sv[i]
                e_i = ev[i]
                lo = jnp.maximum(s_i, c)
                hi = jnp.minimum(e_i, c + chunk)
                jlo = jnp.maximum((lo - c) // LANES, 0)
                jhi = jnp.minimum((hi - c + LANES - 1) // LANES, chunk // LANES)
                jhi = jnp.maximum(jlo, jhi)

                def vb(j, carry, s_i=s_i, e_i=e_i, c=c, buf=buf):
                    off = j * LANES
                    t = (c + off) + iota16
                    x = buf[pl.ds(off, LANES)]
                    d = (t >= s_i) & (t < e_i)
                    buf[pl.ds(off, LANES)] = jnp.where(d, jnp.float32(0.0), x)
                    return carry

                lax.fori_loop(jlo, jhi, vb, 0)

        in_h = {}
        out_h = {}
        waited = set()
        for ci in range(min(nbuf - 1, nch)):
            in_h[ci] = start_in(ci)
        for ci in range(nch):
            in_h[ci].wait()
            mask_chunk(ci)
            out_h[ci] = start_out(ci)
            nxt = ci + nbuf - 1
            if nxt < nch:
                prev = nxt - nbuf
                if prev in out_h:
                    out_h[prev].wait()
                    waited.add(prev)
                in_h[nxt] = start_in(nxt)
        for ci in range(nch):
            if ci not in waited:
                out_h[ci].wait()

    return sc_fn


def kernel(waveforms):
    b, ch, length = waveforms.shape
    batch = b * ch
    starts, ends = _drop_spans(batch, length)
    sb = jnp.zeros((batch, SPAN_PAD), jnp.int32).at[:, :MAX_DROPS].set(starts)
    eb = jnp.zeros((batch, SPAN_PAD), jnp.int32).at[:, :MAX_DROPS].set(ends)
    fn = _build(batch, length)
    out = fn(waveforms.reshape(batch, length), sb, eb)
    return out.reshape(b, ch, length)


# speedup vs baseline: 1.2439x; 1.0085x over previous
import functools

import jax
import jax.numpy as jnp
from jax import lax
from jax.experimental import pallas as pl
from jax.experimental.pallas import tpu as pltpu
from jax.experimental.pallas import tpu_sc as plsc

DROP_LEN_LOW = 1000
DROP_LEN_HIGH = 2000
DROP_CNT_LOW = 1
DROP_CNT_HIGH = 5
P = 0.5
MAX_DROPS = DROP_CNT_HIGH
LANES = 16
SPAN_PAD = 16


def _drop_spans(batch, length):
    key = jax.random.key(42)
    k_apply, k_cnt, k_len, k_start = jax.random.split(key, 4)
    apply_flag = jax.random.uniform(k_apply, ()) <= P
    n_drops = jax.random.randint(k_cnt, (batch,), DROP_CNT_LOW, DROP_CNT_HIGH + 1)
    drop_len = jax.random.randint(
        k_len, (batch, MAX_DROPS), DROP_LEN_LOW, min(DROP_LEN_HIGH + 1, length)
    )
    u = jax.random.uniform(k_start, (batch, MAX_DROPS))
    start_range = jnp.maximum(1, length - drop_len)
    start = jnp.floor(u * start_range).astype(jnp.int32)
    active = (jnp.arange(MAX_DROPS)[None, :] < n_drops[:, None]) & apply_flag
    starts = jnp.where(active, start, 0).astype(jnp.int32)
    ends = jnp.where(active, start + drop_len, 0).astype(jnp.int32)
    return starts, ends


@functools.lru_cache(maxsize=None)
def _build(batch, length):
    info = plsc.get_sparse_core_info()
    nc, ns = info.num_cores, info.num_subcores
    nw = nc * ns
    assert batch == nw, (batch, nw)

    chunk = 32000
    nbuf = 3
    assert length % chunk == 0
    nch = length // chunk

    mesh = plsc.VectorSubcoreMesh(core_axis_name="c", subcore_axis_name="s")

    @functools.partial(
        pl.kernel,
        mesh=mesh,
        out_type=jax.ShapeDtypeStruct((batch, length), jnp.float32),
        scratch_types=[
            pltpu.VMEM((SPAN_PAD,), jnp.int32),
            pltpu.VMEM((SPAN_PAD,), jnp.int32),
        ]
        + [pltpu.VMEM((chunk,), jnp.float32) for _ in range(nbuf)]
        + [pltpu.SemaphoreType.DMA for _ in range(2 * nbuf)],
    )
    def sc_fn(w_hbm, sb_hbm, eb_hbm, out_hbm, s_vm, e_vm, *rest):
        bufs = rest[:nbuf]
        in_sems = rest[nbuf : 2 * nbuf]
        out_sems = rest[2 * nbuf : 3 * nbuf]
        r = lax.axis_index("s") * nc + lax.axis_index("c")
        pltpu.sync_copy(sb_hbm.at[r], s_vm)
        pltpu.sync_copy(eb_hbm.at[r], e_vm)
        sv = s_vm[...]
        ev = e_vm[...]
        iota16 = lax.iota(jnp.int32, LANES)

        def start_in(ci):
            b = ci % nbuf
            return pltpu.async_copy(
                w_hbm.at[r, pl.ds(ci * chunk, chunk)], bufs[b], in_sems[b]
            )

        def start_out(ci):
            b = ci % nbuf
            return pltpu.async_copy(
                bufs[b], out_hbm.at[r, pl.ds(ci * chunk, chunk)], out_sems[b]
            )

        def mask_chunk(ci):
            b = ci % nbuf
            buf = bufs[b]
            c = ci * chunk
            for i in range(MAX_DROPS):
                s_i = sv[i]
                e_i = ev[i]
                lo = jnp.maximum(s_i, c)
                hi = jnp.minimum(e_i, c + chunk)
                jlo = jnp.maximum((lo - c) // LANES, 0)
                jhi = jnp.minimum((hi - c + LANES - 1) // LANES, chunk // LANES)
                jhi = jnp.maximum(jlo, jhi)

                def vb(j, carry, s_i=s_i, e_i=e_i, c=c, buf=buf):
                    off = j * LANES
                    t = (c + off) + iota16
                    x = buf[pl.ds(off, LANES)]
                    d = (t >= s_i) & (t < e_i)
                    buf[pl.ds(off, LANES)] = jnp.where(d, jnp.float32(0.0), x)
                    return carry

                lax.fori_loop(jlo, jhi, vb, 0)

        in_h = {}
        out_h = {}
        waited = set()
        for ci in range(min(nbuf - 1, nch)):
            in_h[ci] = start_in(ci)
        for ci in range(nch):
            in_h[ci].wait()
            mask_chunk(ci)
            out_h[ci] = start_out(ci)
            nxt = ci + nbuf - 1
            if nxt < nch:
                prev = nxt - nbuf
                if prev in out_h:
                    out_h[prev].wait()
                    waited.add(prev)
                in_h[nxt] = start_in(nxt)
        for ci in range(nch):
            if ci not in waited:
                out_h[ci].wait()

    return sc_fn


def kernel(waveforms):
    b, ch, length = waveforms.shape
    batch = b * ch
    starts, ends = _drop_spans(batch, length)
    sb = jnp.zeros((batch, SPAN_PAD), jnp.int32).at[:, :MAX_DROPS].set(starts)
    eb = jnp.zeros((batch, SPAN_PAD), jnp.int32).at[:, :MAX_DROPS].set(ends)
    fn = _build(batch, length)
    out = fn(waveforms.reshape(batch, length), sb, eb)
    return out.reshape(b, ch, length)


# speedup vs baseline: 3.4901x; 2.8058x over previous
import functools

import jax
import jax.numpy as jnp
import numpy as np
from jax import lax
from jax.experimental import pallas as pl
from jax.experimental.pallas import tpu as pltpu
from jax.experimental.pallas import tpu_sc as plsc

DROP_LEN_LOW = 1000
DROP_LEN_HIGH = 2000
DROP_CNT_LOW = 1
DROP_CNT_HIGH = 5
P = 0.5
MAX_DROPS = DROP_CNT_HIGH
LANES = 16


_ROT = (np.uint32([13, 15, 26, 6]), np.uint32([17, 29, 16, 24]))


def _rotl(x, r):
    r = np.uint32(r)
    return ((x << r) | (x >> np.uint32(32 - r))).astype(np.uint32)


def _tf2x32(k1, k2, x0, x1):
    ks = (np.uint32(k1), np.uint32(k2),
          np.uint32(np.uint32(k1) ^ np.uint32(k2) ^ np.uint32(0x1BD11BDA)))
    a = (x0.astype(np.uint32) + ks[0]).astype(np.uint32)
    b = (x1.astype(np.uint32) + ks[1]).astype(np.uint32)
    x = [a, b]

    def rounds(rs):
        for r in rs:
            x[0] = (x[0] + x[1]).astype(np.uint32)
            x[1] = (_rotl(x[1], r) ^ x[0]).astype(np.uint32)

    inject = ((ks[1], ks[2]), (ks[2], ks[0]), (ks[0], ks[1]),
              (ks[1], ks[2]), (ks[2], ks[0]))
    for i, (ka, kb) in enumerate(inject):
        rounds(_ROT[i % 2])
        x[0] = (x[0] + ka).astype(np.uint32)
        x[1] = (x[1] + kb + np.uint32(i + 1)).astype(np.uint32)
    return x[0], x[1]


def _np_split(k, n):
    b1, b2 = _tf2x32(k[0], k[1], np.zeros(n, np.uint32),
                     np.arange(n, dtype=np.uint32))
    return [(b1[i], b2[i]) for i in range(n)]


def _np_bits(k, shape):
    n = int(np.prod(shape)) if shape else 1
    b1, b2 = _tf2x32(k[0], k[1], np.zeros(n, np.uint32),
                     np.arange(n, dtype=np.uint32))
    out = (b1 ^ b2).astype(np.uint32)
    return out.reshape(shape) if shape else out[0]


def _np_uniform(k, shape):
    bits = np.asarray(_np_bits(k, shape))
    f = (((bits >> np.uint32(9)) | np.uint32(0x3F800000))
         .astype(np.uint32).view(np.float32)) - np.float32(1.0)
    f = np.maximum(np.float32(0.0), f)
    return f.reshape(shape) if shape else np.float32(f)


def _np_randint(k, shape, lo, hi):
    k1, k2 = _np_split(k, 2)
    hb = _np_bits(k1, shape)
    lb = _np_bits(k2, shape)
    span = np.uint32(hi - lo)
    mult = np.uint32(np.uint32(65536) % span)
    mult = np.uint32((np.uint64(mult) * np.uint64(mult)) % np.uint64(span))
    off = (((hb % span) * mult) + (lb % span)) % span
    return (np.int32(lo) + off.astype(np.int32)).astype(np.int32)


@functools.lru_cache(maxsize=None)
def _drop_spans(batch, length):
    key = (np.uint32(0), np.uint32(42))
    k_apply, k_cnt, k_len, k_start = _np_split(key, 4)
    apply_flag = _np_uniform(k_apply, ()) <= np.float32(P)
    n_drops = _np_randint(k_cnt, (batch,), DROP_CNT_LOW, DROP_CNT_HIGH + 1)
    drop_len = _np_randint(
        k_len, (batch, MAX_DROPS), DROP_LEN_LOW, min(DROP_LEN_HIGH + 1, length)
    )
    u = _np_uniform(k_start, (batch, MAX_DROPS))
    start_range = np.maximum(1, length - drop_len).astype(np.float32)
    start = np.floor(u * start_range).astype(np.int32)
    active = (np.arange(MAX_DROPS)[None, :] < n_drops[:, None]) & bool(apply_flag)
    starts = np.where(active, start, 0).astype(np.int32)
    ends = np.where(active, start + drop_len, 0).astype(np.int32)
    return starts, ends


@functools.lru_cache(maxsize=None)
def _build(batch, channels, length, spans):
    info = plsc.get_sparse_core_info()
    nc, ns = info.num_cores, info.num_subcores
    nw = nc * ns
    assert batch * channels == nw, (batch, channels, nw)

    chunk = 16000
    nbuf = 4
    assert length % chunk == 0
    nch = length // chunk

    mesh = plsc.VectorSubcoreMesh(core_axis_name="c", subcore_axis_name="s")

    @functools.partial(
        pl.kernel,
        mesh=mesh,
        out_type=jax.ShapeDtypeStruct((batch, channels, length), jnp.float32),
        scratch_types=[pltpu.VMEM_SHARED((ns, nbuf, chunk), jnp.float32)]
        + [pltpu.SemaphoreType.DMA for _ in range(2 * nbuf)],
    )
    def sc_fn(w_hbm, out_hbm, *rest):
        shared = rest[0]
        sid = lax.axis_index("s")
        bufs = [shared.at[sid, b] for b in range(nbuf)]
        in_sems = rest[1 : 1 + nbuf]
        out_sems = rest[1 + nbuf : 1 + 2 * nbuf]
        r = lax.axis_index("s") * nc + lax.axis_index("c")
        iota16 = lax.iota(jnp.int32, LANES)
        if channels == 1:
            row_i, ch_i = r, 0
        else:
            row_i, ch_i = r // channels, r % channels

        def start_in(ci):
            b = ci % nbuf
            return pltpu.async_copy(
                w_hbm.at[row_i, ch_i, pl.ds(ci * chunk, chunk)],
                bufs[b],
                in_sems[b],
            )

        def start_out(ci):
            b = ci % nbuf
            return pltpu.async_copy(
                bufs[b],
                out_hbm.at[row_i, ch_i, pl.ds(ci * chunk, chunk)],
                out_sems[b],
            )

        def mask_chunk(ci):
            buf = bufs[ci % nbuf]
            c = ci * chunk
            for row, row_spans in enumerate(spans):
                todo = []
                for s, e in row_spans:
                    lo, hi = max(s, c), min(e, c + chunk)
                    if lo < hi:
                        todo.append((s, e, (lo - c) // LANES,
                                     (hi - c + LANES - 1) // LANES))
                if not todo:
                    continue

                @pl.when(r == row)
                def _(todo=todo, buf=buf, c=c):
                    for s, e, jlo, jhi in todo:
                        def vb(j, carry, s=s, e=e, c=c, buf=buf):
                            off = j * LANES
                            t = (c + off) + iota16
                            x = buf[pl.ds(off, LANES)]
                            d = (t >= s) & (t < e)
                            buf[pl.ds(off, LANES)] = jnp.where(
                                d, jnp.float32(0.0), x)
                            return carry

                        lax.fori_loop(jlo, jhi, vb, 0)

        in_h = {}
        out_h = {}
        waited = set()
        for ci in range(min(nbuf - 1, nch)):
            in_h[ci] = start_in(ci)
        for ci in range(nch):
            in_h[ci].wait()
            mask_chunk(ci)
            out_h[ci] = start_out(ci)
            nxt = ci + nbuf - 1
            if nxt < nch:
                prev = nxt - nbuf
                if prev in out_h:
                    out_h[prev].wait()
                    waited.add(prev)
                in_h[nxt] = start_in(nxt)
        for ci in range(nch):
            if ci not in waited:
                out_h[ci].wait()

    return sc_fn


def kernel(waveforms):
    b, ch, length = waveforms.shape
    starts, ends = _drop_spans(b, length)
    spans = tuple(
        tuple((int(s), int(e)) for s, e in zip(starts[row], ends[row]) if s < e)
        for row in range(b)
        for _ in range(ch)
    )
    fn = _build(b, ch, length, spans)
    return fn(waveforms)


# speedup vs baseline: 3.5771x; 1.0249x over previous
import functools

import jax
import jax.numpy as jnp
import numpy as np
from jax import lax
from jax.experimental import pallas as pl
from jax.experimental.pallas import tpu as pltpu
from jax.experimental.pallas import tpu_sc as plsc

DROP_LEN_LOW = 1000
DROP_LEN_HIGH = 2000
DROP_CNT_LOW = 1
DROP_CNT_HIGH = 5
P = 0.5
MAX_DROPS = DROP_CNT_HIGH
LANES = 16


_ROT = (np.uint32([13, 15, 26, 6]), np.uint32([17, 29, 16, 24]))


def _rotl(x, r):
    r = np.uint32(r)
    return ((x << r) | (x >> np.uint32(32 - r))).astype(np.uint32)


def _tf2x32(k1, k2, x0, x1):
    ks = (np.uint32(k1), np.uint32(k2),
          np.uint32(np.uint32(k1) ^ np.uint32(k2) ^ np.uint32(0x1BD11BDA)))
    a = (x0.astype(np.uint32) + ks[0]).astype(np.uint32)
    b = (x1.astype(np.uint32) + ks[1]).astype(np.uint32)
    x = [a, b]

    def rounds(rs):
        for r in rs:
            x[0] = (x[0] + x[1]).astype(np.uint32)
            x[1] = (_rotl(x[1], r) ^ x[0]).astype(np.uint32)

    inject = ((ks[1], ks[2]), (ks[2], ks[0]), (ks[0], ks[1]),
              (ks[1], ks[2]), (ks[2], ks[0]))
    for i, (ka, kb) in enumerate(inject):
        rounds(_ROT[i % 2])
        x[0] = (x[0] + ka).astype(np.uint32)
        x[1] = (x[1] + kb + np.uint32(i + 1)).astype(np.uint32)
    return x[0], x[1]


def _np_split(k, n):
    b1, b2 = _tf2x32(k[0], k[1], np.zeros(n, np.uint32),
                     np.arange(n, dtype=np.uint32))
    return [(b1[i], b2[i]) for i in range(n)]


def _np_bits(k, shape):
    n = int(np.prod(shape)) if shape else 1
    b1, b2 = _tf2x32(k[0], k[1], np.zeros(n, np.uint32),
                     np.arange(n, dtype=np.uint32))
    out = (b1 ^ b2).astype(np.uint32)
    return out.reshape(shape) if shape else out[0]


def _np_uniform(k, shape):
    bits = np.asarray(_np_bits(k, shape))
    f = (((bits >> np.uint32(9)) | np.uint32(0x3F800000))
         .astype(np.uint32).view(np.float32)) - np.float32(1.0)
    f = np.maximum(np.float32(0.0), f)
    return f.reshape(shape) if shape else np.float32(f)


def _np_randint(k, shape, lo, hi):
    k1, k2 = _np_split(k, 2)
    hb = _np_bits(k1, shape)
    lb = _np_bits(k2, shape)
    span = np.uint32(hi - lo)
    mult = np.uint32(np.uint32(65536) % span)
    mult = np.uint32((np.uint64(mult) * np.uint64(mult)) % np.uint64(span))
    off = (((hb % span) * mult) + (lb % span)) % span
    return (np.int32(lo) + off.astype(np.int32)).astype(np.int32)


@functools.lru_cache(maxsize=None)
def _drop_spans(batch, length):
    key = (np.uint32(0), np.uint32(42))
    k_apply, k_cnt, k_len, k_start = _np_split(key, 4)
    apply_flag = _np_uniform(k_apply, ()) <= np.float32(P)
    n_drops = _np_randint(k_cnt, (batch,), DROP_CNT_LOW, DROP_CNT_HIGH + 1)
    drop_len = _np_randint(
        k_len, (batch, MAX_DROPS), DROP_LEN_LOW, min(DROP_LEN_HIGH + 1, length)
    )
    u = _np_uniform(k_start, (batch, MAX_DROPS))
    start_range = np.maximum(1, length - drop_len).astype(np.float32)
    start = np.floor(u * start_range).astype(np.int32)
    active = (np.arange(MAX_DROPS)[None, :] < n_drops[:, None]) & bool(apply_flag)
    starts = np.where(active, start, 0).astype(np.int32)
    ends = np.where(active, start + drop_len, 0).astype(np.int32)
    return starts, ends


@functools.lru_cache(maxsize=None)
def _build(batch, channels, length, spans):
    info = plsc.get_sparse_core_info()
    nc, ns = info.num_cores, info.num_subcores
    nw = nc * ns
    assert batch * channels == nw, (batch, channels, nw)

    chunk = 16000
    nbuf = 4
    assert length % chunk == 0
    nch = length // chunk

    mesh = plsc.VectorSubcoreMesh(core_axis_name="c", subcore_axis_name="s")

    @functools.partial(
        pl.kernel,
        mesh=mesh,
        out_type=jax.ShapeDtypeStruct((batch, channels, length), jnp.float32),
        scratch_types=[pltpu.VMEM((chunk,), jnp.float32) for _ in range(nbuf)]
        + [pltpu.SemaphoreType.DMA for _ in range(2 * nbuf)],
    )
    def sc_fn(w_hbm, out_hbm, *rest):
        bufs = rest[:nbuf]
        in_sems = rest[nbuf : 2 * nbuf]
        out_sems = rest[2 * nbuf : 3 * nbuf]
        r = lax.axis_index("s") * nc + lax.axis_index("c")
        iota16 = lax.iota(jnp.int32, LANES)
        if channels == 1:
            row_i, ch_i = r, 0
        else:
            row_i, ch_i = r // channels, r % channels

        def start_in(ci):
            b = ci % nbuf
            return pltpu.async_copy(
                w_hbm.at[row_i, ch_i, pl.ds(ci * chunk, chunk)],
                bufs[b],
                in_sems[b],
            )

        def start_out(ci):
            b = ci % nbuf
            return pltpu.async_copy(
                bufs[b],
                out_hbm.at[row_i, ch_i, pl.ds(ci * chunk, chunk)],
                out_sems[b],
            )

        def mask_chunk(ci):
            buf = bufs[ci % nbuf]
            c = ci * chunk
            for row, row_spans in enumerate(spans):
                todo = []
                for s, e in row_spans:
                    lo, hi = max(s, c), min(e, c + chunk)
                    if lo < hi:
                        todo.append((s, e, (lo - c) // LANES,
                                     (hi - c + LANES - 1) // LANES))
                if not todo:
                    continue

                @pl.when(r == row)
                def _(todo=todo, buf=buf, c=c):
                    for s, e, jlo, jhi in todo:
                        def vb(j, carry, s=s, e=e, c=c, buf=buf):
                            off = j * LANES
                            t = (c + off) + iota16
                            x = buf[pl.ds(off, LANES)]
                            d = (t >= s) & (t < e)
                            buf[pl.ds(off, LANES)] = jnp.where(
                                d, jnp.float32(0.0), x)
                            return carry

                        lax.fori_loop(jlo, jhi, vb, 0)

        in_h = {}
        out_h = {}
        waited = set()
        for ci in range(min(nbuf - 1, nch)):
            in_h[ci] = start_in(ci)
        for ci in range(nch):
            in_h[ci].wait()
            mask_chunk(ci)
            out_h[ci] = start_out(ci)
            nxt = ci + nbuf - 1
            if nxt < nch:
                prev = nxt - nbuf
                if prev in out_h:
                    out_h[prev].wait()
                    waited.add(prev)
                in_h[nxt] = start_in(nxt)
        for ci in range(nch):
            if ci not in waited:
                out_h[ci].wait()

    return sc_fn


def kernel(waveforms):
    b, ch, length = waveforms.shape
    starts, ends = _drop_spans(b, length)
    spans = tuple(
        tuple((int(s), int(e)) for s, e in zip(starts[row], ends[row]) if s < e)
        for row in range(b)
        for _ in range(ch)
    )
    fn = _build(b, ch, length, spans)
    return fn(waveforms)
